# baseline (speedup 1.0000x reference)
import functools
import math

import jax
import jax.numpy as jnp
from jax import lax
from jax.experimental import pallas as pl
from jax.experimental.pallas import tpu as pltpu
from jax.experimental.pallas import tpu_sc as plsc

DIM = 64
SCALE = math.sqrt(DIM)
NC = 2
NS = 16
NW = NC * NS
LANES = 16
NA = 4096
NB = 200
ABLK = NA // NW
NBUF = 4


@jax.jit
def _sc_embed(ids2d, table2):
    mesh = plsc.VectorSubcoreMesh(core_axis_name="c", subcore_axis_name="s")

    @functools.partial(
        pl.kernel,
        mesh=mesh,
        out_type=jax.ShapeDtypeStruct((NB, 8, NW, 8, 128), jnp.float32),
        scratch_types=(
            [pltpu.VMEM((NB, ABLK), jnp.int32)]
            + [pltpu.VMEM((ABLK, 128), jnp.float32) for _ in range(NBUF)]
            + [pltpu.VMEM((ABLK,), jnp.int32) for _ in range(NBUF)]
            + [pltpu.VMEM((ABLK,), jnp.int32) for _ in range(NBUF)]
            + [pltpu.VMEM((DIM, ABLK), jnp.float32) for _ in range(2)]
            + [pltpu.SemaphoreType.DMA for _ in range(NBUF + 2)]
        ),
        compiler_params=pltpu.CompilerParams(needs_layout_passes=False),
    )
    def k(ids_hbm, table_hbm, out_hbm, idsv, *bufs):
        rows = bufs[:NBUF]
        rowsb = bufs[NBUF:2 * NBUF]
        parb = bufs[2 * NBUF:3 * NBUF]
        outb = bufs[3 * NBUF:3 * NBUF + 2]
        gsem = bufs[3 * NBUF + 2:4 * NBUF + 2]
        ssem = bufs[4 * NBUF + 2:]
        wid = lax.axis_index("s") * NC + lax.axis_index("c")
        pltpu.sync_copy(ids_hbm.at[:, pl.ds(wid * ABLK, ABLK)], idsv)
        iota = lax.iota(jnp.int32, LANES)

        def prologue(b, buf):
            for q in range(ABLK // LANES):
                v = idsv[b, pl.ds(q * LANES, LANES)]
                rowsb[buf][pl.ds(q * LANES, LANES)] = (
                    lax.shift_right_logical(v, 1))
                parb[buf][pl.ds(q * LANES, LANES)] = (v & 1) * DIM

        def gather(buf):
            pltpu.async_copy(table_hbm.at[rowsb[buf]], rows[buf], gsem[buf])

        def wait_gather(buf):
            pltpu.make_async_copy(
                table_hbm.at[pl.ds(0, ABLK)], rows[buf], gsem[buf]).wait()

        def wait_store(ob):
            for gg in range(8):
                pltpu.make_async_copy(
                    outb[ob].at[pl.ds(gg * 8, 8), :],
                    out_hbm.at[0, gg, wid], ssem[ob]).wait()

        for b in range(NBUF - 1):
            prologue(b, b)
            gather(b)

        def outer(gi, carry):
            for kk in range(NBUF):
                g = gi * NBUF + kk
                ob = kk % 2
                wait_gather(kk)
                src = rows[kk]
                dst = outb[ob]
                pv = parb[kk]

                @pl.when(g >= 2)
                def _():
                    wait_store(ob)

                @plsc.parallel_loop(0, DIM * 8, step=1, unroll=4)
                def sweep(j):
                    c = lax.shift_right_logical(j, 3)
                    m = j & 7
                    tok = m * LANES + iota
                    word = pv[pl.ds(m * LANES, LANES)] + c
                    vals = plsc.load_gather(src, [tok, word])
                    dst[c, pl.ds(m * LANES, LANES)] = vals * SCALE

                for gg in range(8):
                    pltpu.async_copy(outb[ob].at[pl.ds(gg * 8, 8), :],
                                     out_hbm.at[g, gg, wid], ssem[ob])

                @pl.when(g + NBUF - 1 < NB)
                def _():
                    bp = (kk - 1) % NBUF
                    prologue(g + NBUF - 1, bp)
                    gather(bp)
            return carry

        lax.fori_loop(0, NB // NBUF, outer, 0)
        wait_store((NB - 2) % 2)
        wait_store((NB - 1) % 2)

    return k(ids2d, table2)


def kernel(token_ids, table):
    ids2d = token_ids.T.astype(jnp.int32)
    table2 = table.reshape(500000, 128)
    out5 = _sc_embed(ids2d, table2)
    return out5.transpose(2, 4, 0, 1, 3).reshape(NA, NB, DIM)
